# baseline (speedup 1.0000x reference)
import functools

import jax
import jax.numpy as jnp
from jax.experimental import pallas as pl
from jax.experimental.pallas import tpu as pltpu

_N_WAY = 5
_K_SHOT = 5
_HW = 25
_TEMP = 2.0
_NEG = -1e30


def _dmn4_kernel(a_ref, s_ref, qy_ref, o_ref, a_scr, b_scr, *, qt, nq):
    bi = pl.program_id(0)
    ti = pl.program_id(1)

    @pl.when((bi == 0) & (ti == 0))
    def _zero():
        o_ref[...] = jnp.zeros((1, 1), jnp.float32)
        a_scr[...] = jnp.zeros(a_scr.shape, jnp.float32)

    @pl.when(ti == 0)
    def _build_support():
        cols = []
        zpad = jnp.zeros((640, 128 - _K_SHOT * _HW), jnp.float32)
        for n in range(_N_WAY):
            for k in range(_K_SHOT):
                cols.append(s_ref[0, n, k])
            cols.append(zpad)
        b_scr[...] = jnp.concatenate(cols, axis=1)

    at = jnp.transpose(a_ref[0], (0, 2, 1))
    for i in range(qt):
        a_scr[i * 32:i * 32 + _HW, :] = at[i]

    a2 = a_scr[...]
    bm = b_scr[...]

    g = jnp.dot(a2, bm, preferred_element_type=jnp.float32)
    qn = jnp.maximum(jnp.sqrt(jnp.sum(a2 * a2, axis=1, keepdims=True)), 1e-12)
    sn = jnp.maximum(jnp.sqrt(jnp.sum(bm * bm, axis=0, keepdims=True)), 1e-12)
    rqn = (1.0 / qn).reshape(qt, 32, 1)
    gn = (g / sn).reshape(qt, 32, 5 * 128)

    lane = jax.lax.broadcasted_iota(jnp.int32, (1, 1, 5 * 128), 2)
    rowi = jax.lax.broadcasted_iota(jnp.int32, (1, 32, 1), 1)
    colvalid = (lane - (lane // 128) * 128) < _K_SHOT * _HW

    sm = jnp.where(colvalid, gn, _NEG)
    maxv = jnp.max(sm, axis=2, keepdims=True)
    jp = jnp.min(jnp.where(sm == maxv, lane, 5 * 128), axis=2, keepdims=True)

    cms = [jnp.max(sm[:, :, n * 128:(n + 1) * 128], axis=2, keepdims=True)
           for n in range(_N_WAY)]

    found = jnp.zeros(maxv.shape, dtype=jnp.bool_)
    second = jnp.full(maxv.shape, _NEG, dtype=jnp.float32)
    for n in range(_N_WAY):
        is_max = cms[n] == maxv
        is_first = is_max & (~found)
        found = found | is_max
        second = jnp.where(is_first, second, jnp.maximum(second, cms[n]))
    diff = (maxv - second) * rqn

    oh = lane == jp
    dm = jnp.where(oh, diff, 0.0)
    colmax = jnp.max(dm, axis=1, keepdims=True)
    wrow = jnp.min(jnp.where(dm == colmax, rowi, 1000), axis=1, keepdims=True)
    mi = jnp.max(jnp.where(oh & (wrow == rowi), 1.0, 0.0), axis=2, keepdims=True)

    logits = [jnp.sum((cms[n] * rqn) * mi, axis=1, keepdims=True) * _TEMP
              for n in range(_N_WAY)]

    qy = qy_ref[0]
    m = logits[0]
    for n in range(1, _N_WAY):
        m = jnp.maximum(m, logits[n])
    se = jnp.zeros(m.shape, jnp.float32)
    sel = jnp.zeros(m.shape, jnp.float32)
    for n in range(_N_WAY):
        se = se + jnp.exp(logits[n] - m)
        sel = sel + jnp.where(qy == n, logits[n], 0.0)
    nll = (m + jnp.log(se)) - sel
    o_ref[...] += jnp.sum(nll, axis=0) / nq


def kernel(support_xf, support_y, query_xf, query_y):
    del support_y
    b, q, c, h, w = query_xf.shape
    hw = h * w
    qt = 25
    nt = q // qt

    a = query_xf.reshape(b, q, c, hw)
    s = support_xf.reshape(b, _N_WAY, _K_SHOT, c, hw)
    qy = query_y.astype(jnp.int32).reshape(b, q, 1, 1)

    out = pl.pallas_call(
        functools.partial(_dmn4_kernel, qt=qt, nq=b * q),
        grid=(b, nt),
        in_specs=[
            pl.BlockSpec((1, qt, c, hw), lambda bi, ti: (bi, ti, 0, 0)),
            pl.BlockSpec((1, _N_WAY, _K_SHOT, c, hw), lambda bi, ti: (bi, 0, 0, 0, 0)),
            pl.BlockSpec((1, qt, 1, 1), lambda bi, ti: (bi, ti, 0, 0)),
        ],
        out_specs=pl.BlockSpec((1, 1), lambda bi, ti: (0, 0)),
        out_shape=jax.ShapeDtypeStruct((1, 1), jnp.float32),
        scratch_shapes=[
            pltpu.VMEM((qt * 32, c), jnp.float32),
            pltpu.VMEM((c, _N_WAY * 128), jnp.float32),
        ],
    )(a, s, qy)
    return out[0, 0]
